# baseline (speedup 1.0000x reference)
import functools

import jax
import jax.numpy as jnp
from jax import lax
from jax.experimental import pallas as pl
from jax.experimental.pallas import tpu as pltpu
from jax.experimental.pallas import tpu_sc as plsc

N = 500000
D = 128
Z = 87

NC = 2
NS = 16
NW = NC * NS

C = 80
N_CHUNKS = N // C
BASE_CHUNKS = N_CHUNKS // NW
EXTRA = N_CHUNKS % NW
TOTAL = BASE_CHUNKS + 1
NIDX = 4


def _table_body(ec_ref, cl_ref, emb_ref, out_ref):
    out_ref[...] = emb_ref[...] + lax.dot_general(
        ec_ref[...],
        cl_ref[...],
        (((1,), (1,)), ((), ())),
        preferred_element_type=jnp.float32,
    )


def _build_table(electron_config, cl, emb_table):
    return pl.pallas_call(
        _table_body,
        out_shape=jax.ShapeDtypeStruct((Z, D), jnp.float32),
    )(electron_config, cl, emb_table)


_mesh = plsc.VectorSubcoreMesh(core_axis_name="c", subcore_axis_name="s")


@functools.partial(
    pl.kernel,
    out_type=jax.ShapeDtypeStruct((N, D), jnp.float32),
    mesh=_mesh,
    scratch_types=[
        pltpu.VMEM((NIDX, C), jnp.int32),
        pltpu.VMEM((NIDX, C, D), jnp.float32),
        pltpu.VMEM_SHARED((Z, D), jnp.float32),
        [pltpu.SemaphoreType.DMA] * NIDX,
        [pltpu.SemaphoreType.DMA] * NIDX,
        [pltpu.SemaphoreType.DMA] * NIDX,
    ],
)
def _gather_kernel(idx_hbm, table_hbm, out_hbm, idx_v, rows_v, table_sh,
                   si, sg, sw):
    sid = lax.axis_index("s")
    wid = sid * NC + lax.axis_index("c")
    n_chunks = BASE_CHUNKS + jnp.where(wid < EXTRA, 1, 0)
    last = wid + (n_chunks - 1) * NW

    @pl.when(sid == 0)
    def _stage():
        pltpu.sync_copy(table_hbm, table_sh)

    plsc.subcore_barrier()

    def off(i):
        return jnp.minimum(wid + i * NW, last) * C

    for s in range(NIDX):
        pltpu.async_copy(idx_hbm.at[pl.ds(off(s), C)], idx_v.at[s], si[s])

    def _wait_idx(s):
        pltpu.make_async_copy(idx_hbm.at[pl.ds(0, C)], idx_v.at[s], si[s]).wait()

    def _wait_write(s):
        pltpu.make_async_copy(rows_v.at[s], out_hbm.at[pl.ds(0, C)], sw[s]).wait()

    for s in range(2):
        _wait_idx(s)
        pltpu.async_copy(table_sh.at[idx_v.at[s]], rows_v.at[s], sg[s])

    def quad(q, carry):
        for s in range(NIDX):
            i = NIDX * q + s
            nxt = (s + 2) % NIDX

            @pl.when(i + 2 < TOTAL)
            def _():
                @pl.when((q > 0) | (s >= NIDX - 2))
                def _():
                    _wait_write(nxt)

                _wait_idx(nxt)
                pltpu.async_copy(
                    table_sh.at[idx_v.at[nxt]], rows_v.at[nxt], sg[nxt]
                )

            pltpu.make_async_copy(
                table_sh.at[idx_v.at[s]], rows_v.at[s], sg[s]
            ).wait()
            pltpu.async_copy(rows_v.at[s], out_hbm.at[pl.ds(off(i), C)], sw[s])

            @pl.when(i + NIDX < TOTAL)
            def _():
                pltpu.async_copy(
                    idx_hbm.at[pl.ds(off(i + NIDX), C)], idx_v.at[s], si[s]
                )
        return carry

    lax.fori_loop(0, TOTAL // NIDX, quad, 0)

    for s in range(NIDX):
        pltpu.make_async_copy(rows_v.at[s], out_hbm.at[pl.ds(0, C)], sw[s]).wait()


def kernel(atomic_numbers, electron_config, emb_table, config_linear):
    table = _build_table(electron_config, config_linear, emb_table)
    return _gather_kernel(atomic_numbers.astype(jnp.int32), table)


# speedup vs baseline: 1.0134x; 1.0134x over previous
import functools

import jax
import jax.numpy as jnp
from jax import lax
from jax.experimental import pallas as pl
from jax.experimental.pallas import tpu as pltpu
from jax.experimental.pallas import tpu_sc as plsc

N = 500000
D = 128
Z = 87

NC = 2
NS = 16
NW = NC * NS

C = 80
N_CHUNKS = N // C
BASE_CHUNKS = N_CHUNKS // NW
EXTRA = N_CHUNKS % NW
TOTAL = BASE_CHUNKS + 1
NIDX = 4


def _table_body(ec_ref, clt_ref, emb_ref, out_ref):
    out_ref[...] = emb_ref[...] + jnp.dot(
        ec_ref[...], clt_ref[...], preferred_element_type=jnp.float32
    )


def _build_table(electron_config, clt, emb_table):
    return pl.pallas_call(
        _table_body,
        out_shape=jax.ShapeDtypeStruct((Z, D), jnp.float32),
    )(electron_config, clt, emb_table)


_mesh = plsc.VectorSubcoreMesh(core_axis_name="c", subcore_axis_name="s")


@functools.partial(
    pl.kernel,
    out_type=jax.ShapeDtypeStruct((N, D), jnp.float32),
    mesh=_mesh,
    scratch_types=[
        pltpu.VMEM((NIDX, C), jnp.int32),
        pltpu.VMEM((NIDX, C, D), jnp.float32),
        pltpu.VMEM_SHARED((Z, D), jnp.float32),
        [pltpu.SemaphoreType.DMA] * NIDX,
        [pltpu.SemaphoreType.DMA] * NIDX,
        [pltpu.SemaphoreType.DMA] * NIDX,
    ],
)
def _gather_kernel(idx_hbm, table_hbm, out_hbm, idx_v, rows_v, table_sh,
                   si, sg, sw):
    sid = lax.axis_index("s")
    wid = sid * NC + lax.axis_index("c")
    n_chunks = BASE_CHUNKS + jnp.where(wid < EXTRA, 1, 0)
    last = wid + (n_chunks - 1) * NW

    @pl.when(sid == 0)
    def _stage():
        pltpu.sync_copy(table_hbm, table_sh)

    plsc.subcore_barrier()

    def off(i):
        return jnp.minimum(wid + i * NW, last) * C

    for s in range(NIDX):
        pltpu.async_copy(idx_hbm.at[pl.ds(off(s), C)], idx_v.at[s], si[s])

    def _wait_idx(s):
        pltpu.make_async_copy(idx_hbm.at[pl.ds(0, C)], idx_v.at[s], si[s]).wait()

    def _wait_write(s):
        pltpu.make_async_copy(rows_v.at[s], out_hbm.at[pl.ds(0, C)], sw[s]).wait()

    for s in range(2):
        _wait_idx(s)
        pltpu.async_copy(table_sh.at[idx_v.at[s]], rows_v.at[s], sg[s])

    def quad(q, carry):
        for s in range(NIDX):
            i = NIDX * q + s
            nxt = (s + 2) % NIDX

            @pl.when(i + 2 < TOTAL)
            def _():
                @pl.when((q > 0) | (s >= NIDX - 2))
                def _():
                    _wait_write(nxt)

                _wait_idx(nxt)
                pltpu.async_copy(
                    table_sh.at[idx_v.at[nxt]], rows_v.at[nxt], sg[nxt]
                )

            pltpu.make_async_copy(
                table_sh.at[idx_v.at[s]], rows_v.at[s], sg[s]
            ).wait()
            pltpu.async_copy(rows_v.at[s], out_hbm.at[pl.ds(off(i), C)], sw[s])

            @pl.when(i + NIDX < TOTAL)
            def _():
                pltpu.async_copy(
                    idx_hbm.at[pl.ds(off(i + NIDX), C)], idx_v.at[s], si[s]
                )
        return carry

    lax.fori_loop(0, TOTAL // NIDX, quad, 0)

    for s in range(NIDX):
        pltpu.make_async_copy(rows_v.at[s], out_hbm.at[pl.ds(0, C)], sw[s]).wait()


def kernel(atomic_numbers, electron_config, emb_table, config_linear):
    table = _build_table(electron_config, config_linear.T, emb_table)
    return _gather_kernel(atomic_numbers.astype(jnp.int32), table)
